# baseline (speedup 1.0000x reference)
import functools

import jax
import jax.numpy as jnp
from jax import lax
from jax.experimental import pallas as pl
from jax.experimental.pallas import tpu as pltpu
from jax.experimental.pallas import tpu_sc as plsc

_NC = 2
_NS = 16
_NW = _NC * _NS
_PW = 128
_LANES = 16
_D2 = 128


def _sc_body(feat_hbm, pids_hbm, ptable_hbm, out_hbm,
             idx_v, emb_v, feat_v, out_v, sem_g, sem_f, sem_o,
             pairs_per_tile):
    wid = lax.axis_index("s") * _NC + lax.axis_index("c")
    base = wid * pairs_per_tile
    n_chunks = pairs_per_tile // _PW

    def prefetch(ci, bb):
        p0 = base + ci * _PW
        pltpu.sync_copy(pids_hbm.at[pl.ds(p0, _PW)], idx_v.at[bb])
        pltpu.async_copy(ptable_hbm.at[idx_v.at[bb]], emb_v.at[bb],
                         sem_g.at[bb])
        pltpu.async_copy(feat_hbm.at[pl.ds(p0, _PW)], feat_v.at[bb],
                         sem_f.at[bb])

    for bb in range(2):
        prefetch(bb, bb)

    @pl.loop(0, n_chunks // 2)
    def _(ci2):
        for bb in range(2):
            ci = ci2 * 2 + bb
            pltpu.make_async_copy(
                feat_hbm.at[pl.ds(0, _PW)], feat_v.at[bb], sem_f.at[bb]
            ).wait()
            pltpu.make_async_copy(
                ptable_hbm.at[idx_v.at[bb]], emb_v.at[bb], sem_g.at[bb]
            ).wait()

            @pl.when(ci >= 2)
            def _():
                pltpu.make_async_copy(
                    out_v.at[bb], out_hbm.at[pl.ds(0, _PW)], sem_o.at[bb]
                ).wait()

            @plsc.parallel_loop(0, _PW, unroll=4)
            def _(j):
                for c in range(0, _D2, _LANES):
                    sl = pl.ds(c, _LANES)
                    out_v[bb, j, sl] = feat_v[bb, j, sl] + emb_v[bb, j, sl]

            pltpu.async_copy(out_v.at[bb],
                             out_hbm.at[pl.ds(base + ci * _PW, _PW)],
                             sem_o.at[bb])

            @pl.when(ci + 2 < n_chunks)
            def _():
                prefetch(ci + 2, bb)

    for bb in range(2):
        pltpu.make_async_copy(
            out_v.at[bb], out_hbm.at[pl.ds(0, _PW)], sem_o.at[bb]
        ).wait()


def kernel(features, modality_ids, modality_table):
    b, s, d = features.shape
    n = b * s
    n2 = n // 2
    pairs_per_tile = n2 // _NW
    feat2 = features.reshape(n2, 2 * d)
    ids = modality_ids.reshape(n).astype(jnp.int32)
    pids = ids[0::2] * modality_table.shape[0] + ids[1::2]
    ptable = jnp.concatenate(
        [jnp.repeat(modality_table, modality_table.shape[0], axis=0),
         jnp.tile(modality_table, (modality_table.shape[0], 1))], axis=1)

    mesh = plsc.VectorSubcoreMesh(core_axis_name="c", subcore_axis_name="s")
    sc_call = functools.partial(_sc_body, pairs_per_tile=pairs_per_tile)
    out = pl.kernel(
        sc_call,
        mesh=mesh,
        out_type=jax.ShapeDtypeStruct((n2, 2 * d), jnp.float32),
        scratch_types=[
            pltpu.VMEM((2, _PW), jnp.int32),
            pltpu.VMEM((2, _PW, _D2), jnp.float32),
            pltpu.VMEM((2, _PW, _D2), jnp.float32),
            pltpu.VMEM((2, _PW, _D2), jnp.float32),
            pltpu.SemaphoreType.DMA((2,)),
            pltpu.SemaphoreType.DMA((2,)),
            pltpu.SemaphoreType.DMA((2,)),
        ],
    )(feat2, pids, ptable)
    return out.reshape(b, s, d)


# speedup vs baseline: 1.1956x; 1.1956x over previous
import functools

import jax
import jax.numpy as jnp
from jax import lax
from jax.experimental import pallas as pl
from jax.experimental.pallas import tpu as pltpu
from jax.experimental.pallas import tpu_sc as plsc

_NC = 2
_NS = 16
_NW = _NC * _NS
_PW = 128
_LANES = 16
_D2 = 128


def _sc_body(feat_hbm, pids_hbm, ptable_hbm, out_hbm,
             idx_v, emb_v, feat_v, out_v, ptable_sh, sem_g, sem_f, sem_o,
             pairs_per_tile):
    wid = lax.axis_index("s") * _NC + lax.axis_index("c")
    base = wid * pairs_per_tile
    n_chunks = pairs_per_tile // _PW

    @pl.when(lax.axis_index("s") == 0)
    def _():
        pltpu.sync_copy(ptable_hbm, ptable_sh)

    plsc.subcore_barrier()

    def prefetch(ci, bb):
        p0 = base + ci * _PW
        pltpu.sync_copy(pids_hbm.at[pl.ds(p0, _PW)], idx_v.at[bb])
        pltpu.async_copy(ptable_sh.at[idx_v.at[bb]], emb_v.at[bb],
                         sem_g.at[bb])
        pltpu.async_copy(feat_hbm.at[pl.ds(p0, _PW)], feat_v.at[bb],
                         sem_f.at[bb])

    for bb in range(2):
        prefetch(bb, bb)

    @pl.loop(0, n_chunks // 2)
    def _(ci2):
        for bb in range(2):
            ci = ci2 * 2 + bb
            pltpu.make_async_copy(
                feat_hbm.at[pl.ds(0, _PW)], feat_v.at[bb], sem_f.at[bb]
            ).wait()
            pltpu.make_async_copy(
                ptable_sh.at[idx_v.at[bb]], emb_v.at[bb], sem_g.at[bb]
            ).wait()

            @pl.when(ci >= 2)
            def _():
                pltpu.make_async_copy(
                    out_v.at[bb], out_hbm.at[pl.ds(0, _PW)], sem_o.at[bb]
                ).wait()

            @plsc.parallel_loop(0, _PW, unroll=4)
            def _(j):
                for c in range(0, _D2, _LANES):
                    sl = pl.ds(c, _LANES)
                    out_v[bb, j, sl] = feat_v[bb, j, sl] + emb_v[bb, j, sl]

            pltpu.async_copy(out_v.at[bb],
                             out_hbm.at[pl.ds(base + ci * _PW, _PW)],
                             sem_o.at[bb])

            @pl.when(ci + 2 < n_chunks)
            def _():
                prefetch(ci + 2, bb)

    for bb in range(2):
        pltpu.make_async_copy(
            out_v.at[bb], out_hbm.at[pl.ds(0, _PW)], sem_o.at[bb]
        ).wait()


def kernel(features, modality_ids, modality_table):
    b, s, d = features.shape
    n = b * s
    n2 = n // 2
    pairs_per_tile = n2 // _NW
    feat2 = features.reshape(n2, 2 * d)
    ids = modality_ids.reshape(n).astype(jnp.int32)
    pids = ids[0::2] * modality_table.shape[0] + ids[1::2]
    ptable = jnp.concatenate(
        [jnp.repeat(modality_table, modality_table.shape[0], axis=0),
         jnp.tile(modality_table, (modality_table.shape[0], 1))], axis=1)

    mesh = plsc.VectorSubcoreMesh(core_axis_name="c", subcore_axis_name="s")
    sc_call = functools.partial(_sc_body, pairs_per_tile=pairs_per_tile)
    out = pl.kernel(
        sc_call,
        mesh=mesh,
        out_type=jax.ShapeDtypeStruct((n2, 2 * d), jnp.float32),
        scratch_types=[
            pltpu.VMEM((2, _PW), jnp.int32),
            pltpu.VMEM((2, _PW, _D2), jnp.float32),
            pltpu.VMEM((2, _PW, _D2), jnp.float32),
            pltpu.VMEM((2, _PW, _D2), jnp.float32),
            pltpu.VMEM_SHARED((256, _D2), jnp.float32),
            pltpu.SemaphoreType.DMA((2,)),
            pltpu.SemaphoreType.DMA((2,)),
            pltpu.SemaphoreType.DMA((2,)),
        ],
    )(feat2, pids, ptable)
    return out.reshape(b, s, d)


# speedup vs baseline: 1.1960x; 1.0004x over previous
import functools

import jax
import jax.numpy as jnp
from jax import lax
from jax.experimental import pallas as pl
from jax.experimental.pallas import tpu as pltpu
from jax.experimental.pallas import tpu_sc as plsc

_NC = 2
_NS = 16
_NW = _NC * _NS
_PW = 128
_LANES = 16
_D2 = 128


def _sc_body(feat_hbm, pids_hbm, ptable_hbm, out_hbm,
             idx_v, emb_v, feat_v, out_v, ptable_sh, sem_g, sem_f, sem_o,
             pairs_per_tile):
    wid = lax.axis_index("s") * _NC + lax.axis_index("c")
    base = wid * pairs_per_tile
    n_chunks = pairs_per_tile // _PW

    @pl.when(lax.axis_index("s") == 0)
    def _():
        pltpu.sync_copy(ptable_hbm, ptable_sh)

    plsc.subcore_barrier()

    def prefetch(ci, bb):
        p0 = base + ci * _PW
        pltpu.sync_copy(pids_hbm.at[pl.ds(p0, _PW)], idx_v.at[bb])
        pltpu.async_copy(ptable_sh.at[idx_v.at[bb]], emb_v.at[bb],
                         sem_g.at[bb])
        pltpu.async_copy(feat_hbm.at[pl.ds(p0, _PW)], feat_v.at[bb],
                         sem_f.at[bb])

    for bb in range(2):
        prefetch(bb, bb)

    @pl.loop(0, n_chunks // 2)
    def _(ci2):
        for bb in range(2):
            ci = ci2 * 2 + bb
            pltpu.make_async_copy(
                feat_hbm.at[pl.ds(0, _PW)], feat_v.at[bb], sem_f.at[bb]
            ).wait()
            pltpu.make_async_copy(
                ptable_sh.at[idx_v.at[bb]], emb_v.at[bb], sem_g.at[bb]
            ).wait()

            @pl.when(ci >= 2)
            def _():
                pltpu.make_async_copy(
                    out_v.at[bb], out_hbm.at[pl.ds(0, _PW)], sem_o.at[bb]
                ).wait()

            @plsc.parallel_loop(0, _PW, unroll=8)
            def _(j):
                for c in range(0, _D2, _LANES):
                    sl = pl.ds(c, _LANES)
                    out_v[bb, j, sl] = feat_v[bb, j, sl] + emb_v[bb, j, sl]

            pltpu.async_copy(out_v.at[bb],
                             out_hbm.at[pl.ds(base + ci * _PW, _PW)],
                             sem_o.at[bb])

            @pl.when(ci + 2 < n_chunks)
            def _():
                prefetch(ci + 2, bb)

    for bb in range(2):
        pltpu.make_async_copy(
            out_v.at[bb], out_hbm.at[pl.ds(0, _PW)], sem_o.at[bb]
        ).wait()


def kernel(features, modality_ids, modality_table):
    b, s, d = features.shape
    n = b * s
    n2 = n // 2
    pairs_per_tile = n2 // _NW
    feat2 = features.reshape(n2, 2 * d)
    ids = modality_ids.reshape(n).astype(jnp.int32)
    pids = ids[0::2] * modality_table.shape[0] + ids[1::2]
    ptable = jnp.concatenate(
        [jnp.repeat(modality_table, modality_table.shape[0], axis=0),
         jnp.tile(modality_table, (modality_table.shape[0], 1))], axis=1)

    mesh = plsc.VectorSubcoreMesh(core_axis_name="c", subcore_axis_name="s")
    sc_call = functools.partial(_sc_body, pairs_per_tile=pairs_per_tile)
    out = pl.kernel(
        sc_call,
        mesh=mesh,
        out_type=jax.ShapeDtypeStruct((n2, 2 * d), jnp.float32),
        scratch_types=[
            pltpu.VMEM((2, _PW), jnp.int32),
            pltpu.VMEM((2, _PW, _D2), jnp.float32),
            pltpu.VMEM((2, _PW, _D2), jnp.float32),
            pltpu.VMEM((2, _PW, _D2), jnp.float32),
            pltpu.VMEM_SHARED((256, _D2), jnp.float32),
            pltpu.SemaphoreType.DMA((2,)),
            pltpu.SemaphoreType.DMA((2,)),
            pltpu.SemaphoreType.DMA((2,)),
        ],
    )(feat2, pids, ptable)
    return out.reshape(b, s, d)


# speedup vs baseline: 1.2221x; 1.0218x over previous
import functools

import jax
import jax.numpy as jnp
from jax import lax
from jax.experimental import pallas as pl
from jax.experimental.pallas import tpu as pltpu
from jax.experimental.pallas import tpu_sc as plsc

_NC = 2
_NS = 16
_NW = _NC * _NS
_PW = 128
_LANES = 16
_D2 = 128


def _sc_body(feat_hbm, pids_hbm, ptable_hbm, out_hbm,
             idx_v, emb_v, feat_v, out_v, ptable_sh, sem_g, sem_f, sem_o,
             pairs_per_tile):
    wid = lax.axis_index("s") * _NC + lax.axis_index("c")
    base = wid * pairs_per_tile
    n_chunks = pairs_per_tile // _PW

    @pl.when(lax.axis_index("s") == 0)
    def _():
        pltpu.sync_copy(ptable_hbm, ptable_sh)

    plsc.subcore_barrier()

    pltpu.sync_copy(pids_hbm.at[pl.ds(base, pairs_per_tile)], idx_v)

    def prefetch(ci, bb):
        p0 = base + ci * _PW
        pltpu.async_copy(ptable_sh.at[idx_v.at[pl.ds(ci * _PW, _PW)]],
                         emb_v.at[bb], sem_g.at[bb])
        pltpu.async_copy(feat_hbm.at[pl.ds(p0, _PW)], feat_v.at[bb],
                         sem_f.at[bb])

    for bb in range(2):
        prefetch(bb, bb)

    @pl.loop(0, n_chunks // 2)
    def _(ci2):
        for bb in range(2):
            ci = ci2 * 2 + bb
            pltpu.make_async_copy(
                feat_hbm.at[pl.ds(0, _PW)], feat_v.at[bb], sem_f.at[bb]
            ).wait()
            pltpu.make_async_copy(
                ptable_sh.at[idx_v.at[pl.ds(ci * _PW, _PW)]],
                emb_v.at[bb], sem_g.at[bb]
            ).wait()

            @pl.when(ci >= 2)
            def _():
                pltpu.make_async_copy(
                    out_v.at[bb], out_hbm.at[pl.ds(0, _PW)], sem_o.at[bb]
                ).wait()

            @plsc.parallel_loop(0, _PW, unroll=8)
            def _(j):
                for c in range(0, _D2, _LANES):
                    sl = pl.ds(c, _LANES)
                    out_v[bb, j, sl] = feat_v[bb, j, sl] + emb_v[bb, j, sl]

            pltpu.async_copy(out_v.at[bb],
                             out_hbm.at[pl.ds(base + ci * _PW, _PW)],
                             sem_o.at[bb])

            @pl.when(ci + 2 < n_chunks)
            def _():
                prefetch(ci + 2, bb)

    for bb in range(2):
        pltpu.make_async_copy(
            out_v.at[bb], out_hbm.at[pl.ds(0, _PW)], sem_o.at[bb]
        ).wait()


def kernel(features, modality_ids, modality_table):
    b, s, d = features.shape
    n = b * s
    n2 = n // 2
    pairs_per_tile = n2 // _NW
    feat2 = features.reshape(n2, 2 * d)
    ids = modality_ids.reshape(n).astype(jnp.int32)
    pids = ids[0::2] * modality_table.shape[0] + ids[1::2]
    ptable = jnp.concatenate(
        [jnp.repeat(modality_table, modality_table.shape[0], axis=0),
         jnp.tile(modality_table, (modality_table.shape[0], 1))], axis=1)

    mesh = plsc.VectorSubcoreMesh(core_axis_name="c", subcore_axis_name="s")
    sc_call = functools.partial(_sc_body, pairs_per_tile=pairs_per_tile)
    out = pl.kernel(
        sc_call,
        mesh=mesh,
        out_type=jax.ShapeDtypeStruct((n2, 2 * d), jnp.float32),
        scratch_types=[
            pltpu.VMEM((n2 // _NW,), jnp.int32),
            pltpu.VMEM((2, _PW, _D2), jnp.float32),
            pltpu.VMEM((2, _PW, _D2), jnp.float32),
            pltpu.VMEM((2, _PW, _D2), jnp.float32),
            pltpu.VMEM_SHARED((256, _D2), jnp.float32),
            pltpu.SemaphoreType.DMA((2,)),
            pltpu.SemaphoreType.DMA((2,)),
            pltpu.SemaphoreType.DMA((2,)),
        ],
    )(feat2, pids, ptable)
    return out.reshape(b, s, d)


# speedup vs baseline: 1.2231x; 1.0009x over previous
import functools

import jax
import jax.numpy as jnp
from jax import lax
from jax.experimental import pallas as pl
from jax.experimental.pallas import tpu as pltpu
from jax.experimental.pallas import tpu_sc as plsc

_NC = 2
_NS = 16
_NW = _NC * _NS
_PW = 128
_LANES = 16
_D2 = 128


def _sc_body(feat_hbm, pids_hbm, ptable_hbm, out_hbm,
             idx_v, emb_v, feat_v, out_v, ptable_sh, sem_g, sem_f, sem_o,
             pairs_per_tile):
    wid = lax.axis_index("s") * _NC + lax.axis_index("c")
    base = wid * pairs_per_tile
    n_chunks = pairs_per_tile // _PW

    @pl.when(lax.axis_index("s") == 0)
    def _():
        pltpu.sync_copy(ptable_hbm, ptable_sh)

    plsc.subcore_barrier()

    pltpu.sync_copy(pids_hbm.at[pl.ds(base, pairs_per_tile)], idx_v)

    def prefetch(ci, bb):
        p0 = base + ci * _PW
        pltpu.async_copy(ptable_sh.at[idx_v.at[pl.ds(ci * _PW, _PW)]],
                         emb_v.at[bb], sem_g.at[bb])
        pltpu.async_copy(feat_hbm.at[pl.ds(p0, _PW)], feat_v.at[bb],
                         sem_f.at[bb])

    for bb in range(2):
        prefetch(bb, bb)

    @pl.loop(0, n_chunks // 2)
    def _(ci2):
        for bb in range(2):
            ci = ci2 * 2 + bb
            pltpu.make_async_copy(
                feat_hbm.at[pl.ds(0, _PW)], feat_v.at[bb], sem_f.at[bb]
            ).wait()
            pltpu.make_async_copy(
                ptable_sh.at[idx_v.at[pl.ds(ci * _PW, _PW)]],
                emb_v.at[bb], sem_g.at[bb]
            ).wait()

            @pl.when(ci >= 2)
            def _():
                pltpu.make_async_copy(
                    out_v.at[bb], out_hbm.at[pl.ds(0, _PW)], sem_o.at[bb]
                ).wait()

            @plsc.parallel_loop(0, _PW, unroll=8)
            def _(j):
                for c in range(0, _D2, _LANES):
                    sl = pl.ds(c, _LANES)
                    out_v[bb, j, sl] = feat_v[bb, j, sl] + emb_v[bb, j, sl]

            pltpu.async_copy(out_v.at[bb],
                             out_hbm.at[pl.ds(base + ci * _PW, _PW)],
                             sem_o.at[bb])

            @pl.when(ci + 2 < n_chunks)
            def _():
                prefetch(ci + 2, bb)

    for bb in range(2):
        pltpu.make_async_copy(
            out_v.at[bb], out_hbm.at[pl.ds(0, _PW)], sem_o.at[bb]
        ).wait()


def kernel(features, modality_ids, modality_table):
    b, s, d = features.shape
    n = b * s
    n2 = n // 2
    pairs_per_tile = n2 // _NW
    feat2 = features.reshape(n2, 2 * d)
    ids = modality_ids.reshape(n).astype(jnp.int32)
    pids = ids[0::2] * modality_table.shape[0] + ids[1::2]
    ptable = jnp.concatenate(
        [jnp.repeat(modality_table, modality_table.shape[0], axis=0),
         jnp.tile(modality_table, (modality_table.shape[0], 1))], axis=1)

    mesh = plsc.VectorSubcoreMesh(core_axis_name="c", subcore_axis_name="s")
    sc_call = functools.partial(_sc_body, pairs_per_tile=pairs_per_tile)
    out = pl.kernel(
        sc_call,
        mesh=mesh,
        compiler_params=pltpu.CompilerParams(use_tc_tiling_on_sc=True),
        out_type=jax.ShapeDtypeStruct((n2, 2 * d), jnp.float32),
        scratch_types=[
            pltpu.VMEM((n2 // _NW,), jnp.int32),
            pltpu.VMEM((2, _PW, _D2), jnp.float32),
            pltpu.VMEM((2, _PW, _D2), jnp.float32),
            pltpu.VMEM((2, _PW, _D2), jnp.float32),
            pltpu.VMEM_SHARED((256, _D2), jnp.float32),
            pltpu.SemaphoreType.DMA((2,)),
            pltpu.SemaphoreType.DMA((2,)),
            pltpu.SemaphoreType.DMA((2,)),
        ],
    )(feat2, pids, ptable)
    return out.reshape(b, s, d)


# speedup vs baseline: 1.2233x; 1.0001x over previous
import functools

import jax
import jax.numpy as jnp
from jax import lax
from jax.experimental import pallas as pl
from jax.experimental.pallas import tpu as pltpu
from jax.experimental.pallas import tpu_sc as plsc

_NC = 2
_NS = 16
_NW = _NC * _NS
_PW = 128
_LANES = 16
_D2 = 128


def _sc_body(feat_hbm, pids_hbm, ptable_hbm, out_hbm,
             idx_v, emb_v, feat_v, out_v, ptable_sh, sem_g, sem_f, sem_o,
             pairs_per_tile):
    wid = lax.axis_index("s") * _NC + lax.axis_index("c")
    base = wid * pairs_per_tile
    n_chunks = pairs_per_tile // _PW

    @pl.when(lax.axis_index("s") == 0)
    def _():
        pltpu.sync_copy(ptable_hbm, ptable_sh)

    plsc.subcore_barrier()

    pltpu.sync_copy(pids_hbm.at[pl.ds(base, pairs_per_tile)], idx_v)

    def prefetch(ci, bb):
        p0 = base + ci * _PW
        pltpu.async_copy(ptable_sh.at[idx_v.at[pl.ds(ci * _PW, _PW)]],
                         emb_v.at[bb], sem_g.at[bb])
        pltpu.async_copy(feat_hbm.at[pl.ds(p0, _PW)], feat_v.at[bb],
                         sem_f.at[bb])

    for bb in range(2):
        prefetch(bb, bb)

    @pl.loop(0, n_chunks // 2)
    def _(ci2):
        for bb in range(2):
            ci = ci2 * 2 + bb
            pltpu.make_async_copy(
                feat_hbm.at[pl.ds(0, _PW)], feat_v.at[bb], sem_f.at[bb]
            ).wait()
            pltpu.make_async_copy(
                ptable_sh.at[idx_v.at[pl.ds(ci * _PW, _PW)]],
                emb_v.at[bb], sem_g.at[bb]
            ).wait()

            @pl.when(ci >= 2)
            def _():
                pltpu.make_async_copy(
                    out_v.at[bb], out_hbm.at[pl.ds(0, _PW)], sem_o.at[bb]
                ).wait()

            @plsc.parallel_loop(0, _PW, unroll=8)
            def _(j):
                for c in range(0, _D2, _LANES):
                    sl = pl.ds(c, _LANES)
                    out_v[bb, j, sl] = feat_v[bb, j, sl] + emb_v[bb, j, sl]

            pltpu.async_copy(out_v.at[bb],
                             out_hbm.at[pl.ds(base + ci * _PW, _PW)],
                             sem_o.at[bb])

            @pl.when(ci + 2 < n_chunks)
            def _():
                prefetch(ci + 2, bb)

    for bb in range(2):
        pltpu.make_async_copy(
            out_v.at[bb], out_hbm.at[pl.ds(0, _PW)], sem_o.at[bb]
        ).wait()


def kernel(features, modality_ids, modality_table):
    b, s, d = features.shape
    n = b * s
    n2 = n // 2
    pairs_per_tile = n2 // _NW
    feat2 = features.reshape(n2, 2 * d)
    ids = modality_ids.reshape(n).astype(jnp.int32)
    pids = ids[0::2] * modality_table.shape[0] + ids[1::2]
    ptable = jnp.concatenate(
        [jnp.repeat(modality_table, modality_table.shape[0], axis=0),
         jnp.tile(modality_table, (modality_table.shape[0], 1))], axis=1)

    mesh = plsc.VectorSubcoreMesh(core_axis_name="c", subcore_axis_name="s")
    sc_call = functools.partial(_sc_body, pairs_per_tile=pairs_per_tile)
    out = pl.kernel(
        sc_call,
        mesh=mesh,
        out_type=jax.ShapeDtypeStruct((n2, 2 * d), jnp.float32),
        scratch_types=[
            pltpu.VMEM((n2 // _NW,), jnp.int32),
            pltpu.VMEM((2, _PW, _D2), jnp.float32),
            pltpu.VMEM((2, _PW, _D2), jnp.float32),
            pltpu.VMEM((2, _PW, _D2), jnp.float32),
            pltpu.VMEM_SHARED((256, _D2), jnp.float32),
            pltpu.SemaphoreType.DMA((2,)),
            pltpu.SemaphoreType.DMA((2,)),
            pltpu.SemaphoreType.DMA((2,)),
        ],
    )(feat2, pids, ptable)
    return out.reshape(b, s, d)
